# baseline (speedup 1.0000x reference)
import jax
import jax.numpy as jnp
from jax.experimental import pallas as pl
from jax.experimental.pallas import tpu as pltpu

B = 4
S = 8192
D = 4096


def _body(hs_hbm, mask_hbm, out_ref, mask_v, msem, ssem, csem):
  mc = pltpu.make_async_copy(mask_hbm, mask_v, msem)
  mc.start()
  spec = []
  for b in range(B):
    c = pltpu.make_async_copy(
        hs_hbm.at[b, pl.ds(S - 1, 1), :], out_ref.at[pl.ds(b, 1), :], ssem)
    c.start()
    spec.append(c)
  mc.wait()

  sums = jnp.maximum(jnp.sum(mask_v[...], axis=1) - 1, 0)
  for c in spec:
    c.wait()

  for b in range(B):
    idx = sums[b]

    @pl.when(idx != S - 1)
    def _fix(b=b, idx=idx):
      c = pltpu.make_async_copy(
          hs_hbm.at[b, pl.ds(idx, 1), :], out_ref.at[pl.ds(b, 1), :], csem)
      c.start()
      c.wait()


@jax.jit
def _pooled(hidden_states, mask):
  f = pl.pallas_call(
      _body,
      out_shape=jax.ShapeDtypeStruct((B, D), jnp.float32),
      in_specs=[
          pl.BlockSpec(memory_space=pl.ANY),
          pl.BlockSpec(memory_space=pl.ANY),
      ],
      out_specs=pl.BlockSpec((B, D), lambda: (0, 0)),
      scratch_shapes=[
          pltpu.VMEM((B, S), jnp.int32),
          pltpu.SemaphoreType.DMA,
          pltpu.SemaphoreType.DMA,
          pltpu.SemaphoreType.DMA,
      ],
  )
  return f(hidden_states, mask)


def kernel(hidden_states, mask):
  return _pooled(hidden_states, mask.astype(jnp.int32))


# speedup vs baseline: 1.0633x; 1.0633x over previous
import jax
import jax.numpy as jnp
from jax.experimental import pallas as pl
from jax.experimental.pallas import tpu as pltpu

B = 4
S = 8192
D = 4096


def _body(hs_hbm, mask_hbm, out_ref, mask_v, msem, ssem, csem):
  mc = pltpu.make_async_copy(mask_hbm, mask_v, msem)
  mc.start()
  spec = []
  for b in range(B):
    c = pltpu.make_async_copy(
        hs_hbm.at[b, pl.ds(S - 1, 1), :], out_ref.at[pl.ds(b, 1), :], ssem)
    c.start()
    spec.append(c)
  mc.wait()

  sums = jnp.maximum(jnp.sum(mask_v[...], axis=1) - 1, 0)
  idxs = [sums[b] for b in range(B)]
  for c in spec:
    c.wait()

  for b in range(B):
    idx = idxs[b]

    @pl.when(idx != S - 1)
    def _fix(b=b, idx=idx):
      c = pltpu.make_async_copy(
          hs_hbm.at[b, pl.ds(idx, 1), :], out_ref.at[pl.ds(b, 1), :], csem)
      c.start()
      c.wait()


@jax.jit
def _pooled(hidden_states, mask):
  f = pl.pallas_call(
      _body,
      out_shape=jax.ShapeDtypeStruct((B, D), jnp.float32),
      in_specs=[
          pl.BlockSpec(memory_space=pl.ANY),
          pl.BlockSpec(memory_space=pl.ANY),
      ],
      out_specs=pl.BlockSpec((B, D), lambda: (0, 0)),
      scratch_shapes=[
          pltpu.VMEM((B, S), jnp.int32),
          pltpu.SemaphoreType.DMA,
          pltpu.SemaphoreType.DMA,
          pltpu.SemaphoreType.DMA,
      ],
  )
  return f(hidden_states, mask)


def kernel(hidden_states, mask):
  return _pooled(hidden_states, mask.astype(jnp.int32))
